# baseline (speedup 1.0000x reference)
import functools

import jax
import jax.numpy as jnp
from jax import lax
from jax.experimental import pallas as pl
from jax.experimental.pallas import tpu as pltpu
from jax.experimental.pallas import tpu_sc as plsc

NUM_LOCATION = 100000
EMBED_DIM = 32
BATCH = 16384
N_COLS = 4

_INFO = plsc.get_sparse_core_info()
NUM_CORES = _INFO.num_cores
NUM_SUBCORES = _INFO.num_subcores
LANES = _INFO.num_lanes
NUM_WORKERS = NUM_CORES * NUM_SUBCORES

B_PER_W = BATCH // NUM_WORKERS
IDX_CHUNK = 128
N_CHUNKS = B_PER_W // IDX_CHUNK


def _body(uf_hbm, table_hbm, out_hbm, uf_v, idx_v, rows_v, sem):
    wid = lax.axis_index("s") * NUM_CORES + lax.axis_index("c")
    base = wid * B_PER_W

    pltpu.sync_copy(uf_hbm.at[pl.ds(base * N_COLS, B_PER_W * N_COLS)], uf_v)

    for i in range(B_PER_W // LANES):
        flat = (lax.iota(jnp.int32, LANES) + i * LANES) * N_COLS + 1
        vals = plsc.load_gather(uf_v, [flat])
        idx_v[i // (IDX_CHUNK // LANES),
              pl.ds((i % (IDX_CHUNK // LANES)) * LANES, LANES)] = vals

    copies = []
    for j in range(N_CHUNKS):
        copies.append(
            pltpu.async_copy(
                table_hbm.at[idx_v.at[j]],
                rows_v.at[pl.ds(j * IDX_CHUNK, IDX_CHUNK)],
                sem,
            )
        )
    for c in copies:
        c.wait()

    pltpu.sync_copy(rows_v, out_hbm.at[pl.ds(base, B_PER_W)])


@jax.jit
def kernel(user_fea, embedding_location):
    mesh = plsc.VectorSubcoreMesh(core_axis_name="c", subcore_axis_name="s")
    run = functools.partial(
        pl.kernel,
        out_type=jax.ShapeDtypeStruct((BATCH, EMBED_DIM), jnp.float32),
        mesh=mesh,
        compiler_params=pltpu.CompilerParams(
            needs_layout_passes=False, use_tc_tiling_on_sc=False
        ),
        scratch_types=[
            pltpu.VMEM((B_PER_W * N_COLS,), jnp.int32),
            pltpu.VMEM((N_CHUNKS, IDX_CHUNK), jnp.int32),
            pltpu.VMEM((B_PER_W, EMBED_DIM), jnp.float32),
            pltpu.SemaphoreType.DMA,
        ],
    )(_body)
    return run(user_fea.reshape(-1), embedding_location)


# speedup vs baseline: 2.1780x; 2.1780x over previous
import functools

import jax
import jax.numpy as jnp
from jax import lax
from jax.experimental import pallas as pl
from jax.experimental.pallas import tpu as pltpu
from jax.experimental.pallas import tpu_sc as plsc

NUM_LOCATION = 100000
EMBED_DIM = 32
BATCH = 16384
N_COLS = 4

_INFO = plsc.get_sparse_core_info()
NUM_CORES = _INFO.num_cores
NUM_SUBCORES = _INFO.num_subcores
LANES = _INFO.num_lanes
NUM_WORKERS = NUM_CORES * NUM_SUBCORES

CHUNK = 1024
N_CHUNKS = BATCH // CHUNK


def _body(uft_hbm, tt_hbm, out_hbm, idx_v, row_v, out_v, sem):
    c = lax.axis_index("s") * NUM_CORES + lax.axis_index("c")

    pltpu.sync_copy(uft_hbm.at[1], idx_v)
    pltpu.sync_copy(tt_hbm.at[c], row_v)

    for ci in range(N_CHUNKS):
        def gather16(k, _):
            idx16 = idx_v[pl.ds(ci * CHUNK + k * LANES, LANES)]
            out_v[pl.ds(k * LANES, LANES)] = plsc.load_gather(row_v, [idx16])
            return 0

        lax.fori_loop(0, CHUNK // LANES, gather16, 0)
        pltpu.sync_copy(out_v, out_hbm.at[c, pl.ds(ci * CHUNK, CHUNK)])


@jax.jit
def kernel(user_fea, embedding_location):
    mesh = plsc.VectorSubcoreMesh(core_axis_name="c", subcore_axis_name="s")
    run = functools.partial(
        pl.kernel,
        out_type=jax.ShapeDtypeStruct((EMBED_DIM, BATCH), jnp.float32),
        mesh=mesh,
        compiler_params=pltpu.CompilerParams(needs_layout_passes=False),
        scratch_types=[
            pltpu.VMEM((BATCH,), jnp.int32),
            pltpu.VMEM((NUM_LOCATION,), jnp.float32),
            pltpu.VMEM((CHUNK,), jnp.float32),
            pltpu.SemaphoreType.DMA,
        ],
    )(_body)
    out_t = run(user_fea.T, embedding_location.T)
    return out_t.T
